# baseline (speedup 1.0000x reference)
import functools

import jax
import jax.numpy as jnp
from jax import lax
from jax.experimental import pallas as pl
from jax.experimental.pallas import tpu as pltpu
from jax.experimental.pallas import tpu_sc as plsc

N = 10000
E = 320000
IN_C = 128
HID = 16
R = 40
NC = 16
NR = N * R
NR_PAD = 409600

NCORES = 2
NSUB = 16
NW = NCORES * NSUB
EW = E // NW
BT = 80
NBATCH = EW // BT

N_PAD = 10240
ROWS_T = N_PAD // NSUB
CNT_T = NR_PAD // NSUB
CNT_CH = 3200
NB_ROWS = 2000
GRID_N = N // NB_ROWS


def _sc_mesh():
    return plsc.VectorSubcoreMesh(core_axis_name="c", subcore_axis_name="s")


_SC_PARAMS = pltpu.CompilerParams(use_tc_tiling_on_sc=False)


def _fill1d(ref, n, val):
    @pl.loop(0, n // 16)
    def _(i):
        ref[pl.ds(i * 16, 16)] = jnp.full((16,), val, ref.dtype)


def _fill2d(ref, rows, val):
    @pl.loop(0, rows)
    def _(i):
        ref[i, :] = jnp.full((16,), val, ref.dtype)


def _msg_prologue(gidx_hbm, dst_hbm, gidx_v, dst_v, tile_v, agg_sh, wid, s):
    pltpu.sync_copy(gidx_hbm.at[wid], gidx_v)
    pltpu.sync_copy(dst_hbm.at[wid], dst_v)
    _fill2d(tile_v, ROWS_T, 0.0)
    row0 = pl.multiple_of(s * ROWS_T, 8)
    pltpu.sync_copy(tile_v, agg_sh.at[pl.ds(row0, ROWS_T), :])
    return row0


def _msg_scale_scatter(rv, norm_v, j, dst_v, agg_sh):
    @pl.loop(0, BT // 16)
    def _(g):
        nv = norm_v[j, pl.ds(g * 16, 16)]
        for k in range(16):
            e = g * 16 + k
            rv[e, :] = rv[e, :] * nv[k]

    pltpu.sync_copy(rv, agg_sh.at[dst_v.at[j]], add=True)


def _msg_epilogue(agg_sh, tile_v, agg_hbm, row0, c, s):
    plsc.subcore_barrier()
    pltpu.sync_copy(agg_sh.at[pl.ds(row0, ROWS_T), :], tile_v)
    out_row = pl.multiple_of(c * N_PAD + s * ROWS_T, 8)
    pltpu.sync_copy(tile_v, agg_hbm.at[pl.ds(out_row, ROWS_T), :])


_MSG_SCRATCH1 = [
    pltpu.VMEM((NBATCH, BT), jnp.int32),
    pltpu.VMEM((NBATCH, BT), jnp.int32),
    pltpu.VMEM((NBATCH, BT), jnp.int32),
    pltpu.VMEM((NBATCH, BT), jnp.int32),
    pltpu.VMEM((NBATCH, BT), jnp.int32),
    pltpu.VMEM((NBATCH, BT), jnp.float32),
    pltpu.VMEM((BT, 16), jnp.float32),
    pltpu.VMEM((BT, 16), jnp.float32),
    pltpu.VMEM((ROWS_T, 16), jnp.float32),
    pltpu.VMEM((CNT_CH,), jnp.float32),
    pltpu.VMEM((BT,), jnp.float32),
    pltpu.VMEM_SHARED((N_PAD, 16), jnp.float32),
    pltpu.VMEM_SHARED((NR_PAD,), jnp.float32),
    pltpu.SemaphoreType.DMA,
    pltpu.SemaphoreType.DMA,
    pltpu.SemaphoreType.DMA,
    pltpu.SemaphoreType.DMA,
]


@functools.partial(
    pl.kernel,
    out_type=(
        jax.ShapeDtypeStruct((NCORES * N_PAD, 16), jnp.float32),
        jax.ShapeDtypeStruct((NW, NBATCH, BT), jnp.float32),
    ),
    mesh=_sc_mesh(),
    compiler_params=_SC_PARAMS,
    scratch_types=_MSG_SCRATCH1,
)
def _sc_msg1(gidx_hbm, dst_hbm, cidx_hbm, h_hbm,
             agg_hbm, norm_out_hbm,
             gidx_v, dst_v, cidx_v, cnta_v, cntb_v, norm_v, rows0, rows1,
             tile_v, cbuf_v, ones_v, agg_sh, cnt_sh,
             gsem0, gsem1, nsem0, nsem1):
    c = lax.axis_index("c")
    s = lax.axis_index("s")
    wid = s * NCORES + c

    pltpu.sync_copy(cidx_hbm.at[2 * s], cnta_v)
    pltpu.sync_copy(cidx_hbm.at[2 * s + 1], cntb_v)
    pltpu.sync_copy(cidx_hbm.at[wid], cidx_v)
    _fill1d(cbuf_v, CNT_CH, 0.0)

    @pl.loop(0, CNT_T // CNT_CH)
    def _(k):
        off = pl.multiple_of(s * CNT_T + k * CNT_CH, 8)
        pltpu.sync_copy(cbuf_v, cnt_sh.at[pl.ds(off, CNT_CH)])

    _fill1d(ones_v, BT, 1.0)
    row0 = _msg_prologue(gidx_hbm, dst_hbm, gidx_v, dst_v, tile_v, agg_sh,
                         wid, s)
    plsc.subcore_barrier()

    @pl.loop(0, NBATCH)
    def _(j):
        pltpu.sync_copy(ones_v, cnt_sh.at[cnta_v.at[j]], add=True)
        pltpu.sync_copy(ones_v, cnt_sh.at[cntb_v.at[j]], add=True)

    plsc.subcore_barrier()

    rows = (rows0, rows1)
    gsems = (gsem0, gsem1)
    nsems = (nsem0, nsem1)

    def issue(j, b):
        pltpu.async_copy(h_hbm.at[gidx_v.at[j]], rows[b], gsems[b])
        pltpu.async_copy(cnt_sh.at[cidx_v.at[j]], norm_v.at[j], nsems[b])

    def process(j, b):
        pltpu.make_async_copy(
            h_hbm.at[pl.ds(0, BT), :], rows[b], gsems[b]).wait()
        pltpu.make_async_copy(
            cnt_sh.at[pl.ds(0, BT)], norm_v.at[j], nsems[b]).wait()

        for g in range(BT // 16):
            cnt = norm_v[j, pl.ds(g * 16, 16)]
            norm_v[j, pl.ds(g * 16, 16)] = 1.0 / jnp.maximum(cnt, 1.0)

        _msg_scale_scatter(rows[b], norm_v, j, dst_v, agg_sh)

    issue(0, 0)

    @pl.loop(0, NBATCH - 1, step=2)
    def _(j):
        issue(j + 1, 1)
        process(j, 0)
        issue(j + 2, 0)
        process(j + 1, 1)

    process(NBATCH - 1, 0)

    pltpu.sync_copy(norm_v, norm_out_hbm.at[wid])
    _msg_epilogue(agg_sh, tile_v, agg_hbm, row0, c, s)


_MSG_SCRATCH2 = [
    pltpu.VMEM((NBATCH, BT), jnp.int32),
    pltpu.VMEM((NBATCH, BT), jnp.int32),
    pltpu.VMEM((NBATCH, BT), jnp.float32),
    pltpu.VMEM((BT, 16), jnp.float32),
    pltpu.VMEM((BT, 16), jnp.float32),
    pltpu.VMEM((ROWS_T, 16), jnp.float32),
    pltpu.VMEM_SHARED((N_PAD, 16), jnp.float32),
    pltpu.SemaphoreType.DMA,
    pltpu.SemaphoreType.DMA,
]


@functools.partial(
    pl.kernel,
    out_type=jax.ShapeDtypeStruct((NCORES * N_PAD, 16), jnp.float32),
    mesh=_sc_mesh(),
    compiler_params=_SC_PARAMS,
    scratch_types=_MSG_SCRATCH2,
)
def _sc_msg2(gidx_hbm, dst_hbm, norm_hbm, h_hbm, agg_hbm,
             gidx_v, dst_v, norm_v, rows0, rows1, tile_v, agg_sh,
             gsem0, gsem1):
    c = lax.axis_index("c")
    s = lax.axis_index("s")
    wid = s * NCORES + c

    pltpu.sync_copy(norm_hbm.at[wid], norm_v)
    row0 = _msg_prologue(gidx_hbm, dst_hbm, gidx_v, dst_v, tile_v, agg_sh,
                         wid, s)
    plsc.subcore_barrier()

    rows = (rows0, rows1)
    gsems = (gsem0, gsem1)

    def issue(j, b):
        pltpu.async_copy(h_hbm.at[gidx_v.at[j]], rows[b], gsems[b])

    def process(j, b):
        pltpu.make_async_copy(
            h_hbm.at[pl.ds(0, BT), :], rows[b], gsems[b]).wait()
        _msg_scale_scatter(rows[b], norm_v, j, dst_v, agg_sh)

    issue(0, 0)

    @pl.loop(0, NBATCH - 1, step=2)
    def _(j):
        issue(j + 1, 1)
        process(j, 0)
        issue(j + 2, 0)
        process(j + 1, 1)

    process(NBATCH - 1, 0)
    _msg_epilogue(agg_sh, tile_v, agg_hbm, row0, c, s)


def _edge_body(s_ref, d_ref, t_ref, g_ref, c_ref):
    t = t_ref[...]
    g_ref[...] = s_ref[...] * R + t
    c_ref[...] = d_ref[...] * R + t


def _tc_edge(src2, dst2, et2):
    return pl.pallas_call(
        _edge_body,
        out_shape=(
            jax.ShapeDtypeStruct(src2.shape, jnp.int32),
            jax.ShapeDtypeStruct(src2.shape, jnp.int32),
        ),
    )(src2, dst2, et2)


def _prep_body(x_ref, wf_ref, r_ref, b_ref, h_ref, xr_ref):
    xv = x_ref[...]
    h_ref[...] = jnp.dot(xv, wf_ref[...], preferred_element_type=jnp.float32)
    xr_ref[...] = (
        jnp.dot(xv, r_ref[...], preferred_element_type=jnp.float32) + b_ref[...]
    )


def _tc_prep(x, wflat, root, bias2d):
    k = x.shape[1]
    m = wflat.shape[1]
    return pl.pallas_call(
        _prep_body,
        grid=(GRID_N,),
        in_specs=[
            pl.BlockSpec((NB_ROWS, k), lambda i: (i, 0)),
            pl.BlockSpec((k, m), lambda i: (0, 0)),
            pl.BlockSpec((k, 16), lambda i: (0, 0)),
            pl.BlockSpec((1, 16), lambda i: (0, 0)),
        ],
        out_specs=(
            pl.BlockSpec((NB_ROWS, m), lambda i: (i, 0)),
            pl.BlockSpec((NB_ROWS, 16), lambda i: (i, 0)),
        ),
        out_shape=(
            jax.ShapeDtypeStruct((N, m), jnp.float32),
            jax.ShapeDtypeStruct((N, 16), jnp.float32),
        ),
    )(x, wflat, root, bias2d)


def _mid_body(agg_ref, xr_ref, wf_ref, r_ref, b_ref, h2_ref, hr_ref):
    h = jnp.maximum(agg_ref[0] + agg_ref[1] + xr_ref[...], 0.0)
    h2_ref[...] = jnp.dot(h, wf_ref[...], preferred_element_type=jnp.float32)
    hr_ref[...] = (
        jnp.dot(h, r_ref[...], preferred_element_type=jnp.float32) + b_ref[...]
    )


def _tc_mid(agg3, xr, wflat2, root2, bias2d):
    m = wflat2.shape[1]
    return pl.pallas_call(
        _mid_body,
        grid=(GRID_N,),
        in_specs=[
            pl.BlockSpec((2, NB_ROWS, 16), lambda i: (0, i, 0)),
            pl.BlockSpec((NB_ROWS, 16), lambda i: (i, 0)),
            pl.BlockSpec((16, m), lambda i: (0, 0)),
            pl.BlockSpec((16, 16), lambda i: (0, 0)),
            pl.BlockSpec((1, 16), lambda i: (0, 0)),
        ],
        out_specs=(
            pl.BlockSpec((NB_ROWS, m), lambda i: (i, 0)),
            pl.BlockSpec((NB_ROWS, 16), lambda i: (i, 0)),
        ),
        out_shape=(
            jax.ShapeDtypeStruct((N, m), jnp.float32),
            jax.ShapeDtypeStruct((N, 16), jnp.float32),
        ),
    )(agg3, xr, wflat2, root2, bias2d)


def _fin_body(agg_ref, hr_ref, o_ref):
    v = agg_ref[0] + agg_ref[1] + hr_ref[...]
    m = jnp.max(v, axis=1, keepdims=True)
    ex = jnp.exp(v - m)
    o_ref[...] = (v - m) - jnp.log(jnp.sum(ex, axis=1, keepdims=True))


def _tc_fin(agg3, hr):
    return pl.pallas_call(
        _fin_body,
        grid=(GRID_N,),
        in_specs=[
            pl.BlockSpec((2, NB_ROWS, 16), lambda i: (0, i, 0)),
            pl.BlockSpec((NB_ROWS, 16), lambda i: (i, 0)),
        ],
        out_specs=pl.BlockSpec((NB_ROWS, 16), lambda i: (i, 0)),
        out_shape=jax.ShapeDtypeStruct((N, 16), jnp.float32),
    )(agg3, hr)


def kernel(x, edge_index, edge_type, basis1, comp1, root1, bias1,
           basis2, comp2, root2, bias2):
    src = edge_index[0]
    dst = edge_index[1]

    gidx2, cidx2 = _tc_edge(
        src.reshape(2500, 128), dst.reshape(2500, 128),
        edge_type.reshape(2500, 128))
    gidx3 = gidx2.reshape(NW, NBATCH, BT)
    cidx3 = cidx2.reshape(NW, NBATCH, BT)
    dst3 = dst.reshape(NW, NBATCH, BT)

    wflat1 = jnp.einsum("rb,bio->iro", comp1, basis1).reshape(IN_C, R * HID)
    wflat2 = jnp.einsum("rb,bio->iro", comp2, basis2).reshape(HID, R * NC)

    h1, xr1 = _tc_prep(x, wflat1, root1, bias1.reshape(1, 16))
    agg1, norm3 = _sc_msg1(gidx3, dst3, cidx3, h1.reshape(NR, 16))
    agg1 = agg1.reshape(NCORES, N_PAD, 16)[:, :N, :]

    h2, hr2 = _tc_mid(agg1, xr1, wflat2, root2, bias2.reshape(1, 16))
    agg2 = _sc_msg2(gidx3, dst3, norm3, h2.reshape(NR, 16))
    agg2 = agg2.reshape(NCORES, N_PAD, 16)[:, :N, :]

    return _tc_fin(agg2, hr2)


# speedup vs baseline: 1.2457x; 1.2457x over previous
import functools

import jax
import jax.numpy as jnp
from jax import lax
from jax.experimental import pallas as pl
from jax.experimental.pallas import tpu as pltpu
from jax.experimental.pallas import tpu_sc as plsc

N = 10000
E = 320000
IN_C = 128
HID = 16
R = 40
NC = 16
NR = N * R
NR_PAD = 409600

NCORES = 2
NSUB = 16
NW = NCORES * NSUB
EW = E // NW
BT = 80
NBATCH = EW // BT

N_PAD = 10240
ROWS_T = N_PAD // NSUB
CNT_T = NR_PAD // NSUB
CNT_CH = 3200
NB_ROWS = 2000
GRID_N = N // NB_ROWS


def _sc_mesh():
    return plsc.VectorSubcoreMesh(core_axis_name="c", subcore_axis_name="s")


_SC_PARAMS = pltpu.CompilerParams(use_tc_tiling_on_sc=False)


def _fill1d(ref, n, val):
    @pl.loop(0, n // 16)
    def _(i):
        ref[pl.ds(i * 16, 16)] = jnp.full((16,), val, ref.dtype)


def _fill2d(ref, rows, val):
    @pl.loop(0, rows)
    def _(i):
        ref[i, :] = jnp.full((16,), val, ref.dtype)


def _msg_prologue(gidx_hbm, dst_hbm, gidx_v, dst_v, tile_v, agg_sh, wid, s):
    pltpu.sync_copy(gidx_hbm.at[wid], gidx_v)
    pltpu.sync_copy(dst_hbm.at[wid], dst_v)
    _fill2d(tile_v, ROWS_T, 0.0)
    row0 = pl.multiple_of(s * ROWS_T, 8)
    pltpu.sync_copy(tile_v, agg_sh.at[pl.ds(row0, ROWS_T), :])
    return row0


def _msg_scale_scatter(rv, norm_v, j, dst_v, agg_sh):
    @pl.loop(0, BT // 16)
    def _(g):
        nv = norm_v[j, pl.ds(g * 16, 16)]
        for k in range(16):
            e = g * 16 + k
            rv[e, :] = rv[e, :] * nv[k]

    pltpu.sync_copy(rv, agg_sh.at[dst_v.at[j]], add=True)


def _msg_epilogue(agg_sh, tile_v, agg_hbm, row0, c, s):
    plsc.subcore_barrier()
    pltpu.sync_copy(agg_sh.at[pl.ds(row0, ROWS_T), :], tile_v)
    out_row = pl.multiple_of(c * N_PAD + s * ROWS_T, 8)
    pltpu.sync_copy(tile_v, agg_hbm.at[pl.ds(out_row, ROWS_T), :])


_MSG_SCRATCH1 = [
    pltpu.VMEM((NBATCH, BT), jnp.int32),
    pltpu.VMEM((NBATCH, BT), jnp.int32),
    pltpu.VMEM((NBATCH, BT), jnp.int32),
    pltpu.VMEM((NBATCH, BT), jnp.int32),
    pltpu.VMEM((NBATCH, BT), jnp.int32),
    pltpu.VMEM((NBATCH, BT), jnp.float32),
    pltpu.VMEM((BT, 16), jnp.float32),
    pltpu.VMEM((BT, 16), jnp.float32),
    pltpu.VMEM((ROWS_T, 16), jnp.float32),
    pltpu.VMEM((CNT_CH,), jnp.float32),
    pltpu.VMEM((BT,), jnp.float32),
    pltpu.VMEM_SHARED((N_PAD, 16), jnp.float32),
    pltpu.VMEM_SHARED((NR_PAD,), jnp.float32),
    pltpu.SemaphoreType.DMA,
    pltpu.SemaphoreType.DMA,
    pltpu.SemaphoreType.DMA,
    pltpu.SemaphoreType.DMA,
]


@functools.partial(
    pl.kernel,
    out_type=(
        jax.ShapeDtypeStruct((NCORES * N_PAD, 16), jnp.float32),
        jax.ShapeDtypeStruct((NW, NBATCH, BT), jnp.float32),
    ),
    mesh=_sc_mesh(),
    compiler_params=_SC_PARAMS,
    scratch_types=_MSG_SCRATCH1,
)
def _sc_msg1(gidx_hbm, dst_hbm, cidx_hbm, h_hbm,
             agg_hbm, norm_out_hbm,
             gidx_v, dst_v, cidx_v, cnta_v, cntb_v, norm_v, rows0, rows1,
             tile_v, cbuf_v, ones_v, agg_sh, cnt_sh,
             gsem0, gsem1, nsem0, nsem1):
    c = lax.axis_index("c")
    s = lax.axis_index("s")
    wid = s * NCORES + c

    pltpu.sync_copy(cidx_hbm.at[2 * s], cnta_v)
    pltpu.sync_copy(cidx_hbm.at[2 * s + 1], cntb_v)
    pltpu.sync_copy(cidx_hbm.at[wid], cidx_v)
    _fill1d(cbuf_v, CNT_CH, 0.0)

    @pl.loop(0, CNT_T // CNT_CH)
    def _(k):
        off = pl.multiple_of(s * CNT_T + k * CNT_CH, 8)
        pltpu.sync_copy(cbuf_v, cnt_sh.at[pl.ds(off, CNT_CH)])

    _fill1d(ones_v, BT, 1.0)
    row0 = _msg_prologue(gidx_hbm, dst_hbm, gidx_v, dst_v, tile_v, agg_sh,
                         wid, s)
    plsc.subcore_barrier()

    @pl.loop(0, NBATCH)
    def _(j):
        pltpu.sync_copy(ones_v, cnt_sh.at[cnta_v.at[j]], add=True)
        pltpu.sync_copy(ones_v, cnt_sh.at[cntb_v.at[j]], add=True)

    plsc.subcore_barrier()

    rows = (rows0, rows1)
    gsems = (gsem0, gsem1)
    nsems = (nsem0, nsem1)

    def issue(j, b):
        pltpu.async_copy(h_hbm.at[gidx_v.at[j]], rows[b], gsems[b])
        pltpu.async_copy(cnt_sh.at[cidx_v.at[j]], norm_v.at[j], nsems[b])

    def process(j, b):
        pltpu.make_async_copy(
            h_hbm.at[pl.ds(0, BT), :], rows[b], gsems[b]).wait()
        pltpu.make_async_copy(
            cnt_sh.at[pl.ds(0, BT)], norm_v.at[j], nsems[b]).wait()

        for g in range(BT // 16):
            cnt = norm_v[j, pl.ds(g * 16, 16)]
            norm_v[j, pl.ds(g * 16, 16)] = 1.0 / jnp.maximum(cnt, 1.0)

        _msg_scale_scatter(rows[b], norm_v, j, dst_v, agg_sh)

    issue(0, 0)

    @pl.loop(0, NBATCH - 1, step=2)
    def _(j):
        issue(j + 1, 1)
        process(j, 0)
        issue(j + 2, 0)
        process(j + 1, 1)

    process(NBATCH - 1, 0)

    pltpu.sync_copy(norm_v, norm_out_hbm.at[wid])
    _msg_epilogue(agg_sh, tile_v, agg_hbm, row0, c, s)


_MSG_SCRATCH2 = [
    pltpu.VMEM((NBATCH, BT), jnp.int32),
    pltpu.VMEM((NBATCH, BT), jnp.int32),
    pltpu.VMEM((NBATCH, BT), jnp.float32),
    pltpu.VMEM((BT, 16), jnp.float32),
    pltpu.VMEM((BT, 16), jnp.float32),
    pltpu.VMEM((ROWS_T, 16), jnp.float32),
    pltpu.VMEM_SHARED((N_PAD, 16), jnp.float32),
    pltpu.SemaphoreType.DMA,
    pltpu.SemaphoreType.DMA,
]


@functools.partial(
    pl.kernel,
    out_type=jax.ShapeDtypeStruct((NCORES * N_PAD, 16), jnp.float32),
    mesh=_sc_mesh(),
    compiler_params=_SC_PARAMS,
    scratch_types=_MSG_SCRATCH2,
)
def _sc_msg2(gidx_hbm, dst_hbm, norm_hbm, h_hbm, agg_hbm,
             gidx_v, dst_v, norm_v, rows0, rows1, tile_v, agg_sh,
             gsem0, gsem1):
    c = lax.axis_index("c")
    s = lax.axis_index("s")
    wid = s * NCORES + c

    pltpu.sync_copy(norm_hbm.at[wid], norm_v)
    row0 = _msg_prologue(gidx_hbm, dst_hbm, gidx_v, dst_v, tile_v, agg_sh,
                         wid, s)
    plsc.subcore_barrier()

    rows = (rows0, rows1)
    gsems = (gsem0, gsem1)

    def issue(j, b):
        pltpu.async_copy(h_hbm.at[gidx_v.at[j]], rows[b], gsems[b])

    def process(j, b):
        pltpu.make_async_copy(
            h_hbm.at[pl.ds(0, BT), :], rows[b], gsems[b]).wait()
        _msg_scale_scatter(rows[b], norm_v, j, dst_v, agg_sh)

    issue(0, 0)

    @pl.loop(0, NBATCH - 1, step=2)
    def _(j):
        issue(j + 1, 1)
        process(j, 0)
        issue(j + 2, 0)
        process(j + 1, 1)

    process(NBATCH - 1, 0)
    _msg_epilogue(agg_sh, tile_v, agg_hbm, row0, c, s)


def _edge_body(s_ref, d_ref, t_ref, g_ref, c_ref):
    t = t_ref[...]
    g_ref[...] = (t // 8) * (8 * N) + s_ref[...] * 8 + (t % 8)
    c_ref[...] = d_ref[...] * R + t


def _tc_edge(src2, dst2, et2):
    return pl.pallas_call(
        _edge_body,
        out_shape=(
            jax.ShapeDtypeStruct(src2.shape, jnp.int32),
            jax.ShapeDtypeStruct(src2.shape, jnp.int32),
        ),
    )(src2, dst2, et2)


def _prep_body(x_ref, wf_ref, r_ref, b_ref, h_ref, xr_ref):
    xv = x_ref[...]
    wf = wf_ref[...]
    for tc in range(5):
        h_ref[tc] = jnp.dot(xv, wf[:, tc * 128:(tc + 1) * 128],
                            preferred_element_type=jnp.float32)
    xr_ref[...] = (
        jnp.dot(xv, r_ref[...], preferred_element_type=jnp.float32) + b_ref[...]
    )


def _tc_prep(x, wflat, root, bias2d):
    k = x.shape[1]
    m = wflat.shape[1]
    return pl.pallas_call(
        _prep_body,
        grid=(GRID_N,),
        in_specs=[
            pl.BlockSpec((NB_ROWS, k), lambda i: (i, 0)),
            pl.BlockSpec((k, m), lambda i: (0, 0)),
            pl.BlockSpec((k, 16), lambda i: (0, 0)),
            pl.BlockSpec((1, 16), lambda i: (0, 0)),
        ],
        out_specs=(
            pl.BlockSpec((5, NB_ROWS, 128), lambda i: (0, i, 0)),
            pl.BlockSpec((NB_ROWS, 16), lambda i: (i, 0)),
        ),
        out_shape=(
            jax.ShapeDtypeStruct((5, N, 128), jnp.float32),
            jax.ShapeDtypeStruct((N, 16), jnp.float32),
        ),
    )(x, wflat, root, bias2d)


def _mid_body(agg_ref, xr_ref, wf_ref, r_ref, b_ref, h2_ref, hr_ref):
    h = jnp.maximum(agg_ref[0] + agg_ref[1] + xr_ref[...], 0.0)
    wf = wf_ref[...]
    for tc in range(5):
        h2_ref[tc] = jnp.dot(h, wf[:, tc * 128:(tc + 1) * 128],
                             preferred_element_type=jnp.float32)
    hr_ref[...] = (
        jnp.dot(h, r_ref[...], preferred_element_type=jnp.float32) + b_ref[...]
    )


def _tc_mid(agg3, xr, wflat2, root2, bias2d):
    m = wflat2.shape[1]
    return pl.pallas_call(
        _mid_body,
        grid=(GRID_N,),
        in_specs=[
            pl.BlockSpec((2, NB_ROWS, 16), lambda i: (0, i, 0)),
            pl.BlockSpec((NB_ROWS, 16), lambda i: (i, 0)),
            pl.BlockSpec((16, m), lambda i: (0, 0)),
            pl.BlockSpec((16, 16), lambda i: (0, 0)),
            pl.BlockSpec((1, 16), lambda i: (0, 0)),
        ],
        out_specs=(
            pl.BlockSpec((5, NB_ROWS, 128), lambda i: (0, i, 0)),
            pl.BlockSpec((NB_ROWS, 16), lambda i: (i, 0)),
        ),
        out_shape=(
            jax.ShapeDtypeStruct((5, N, 128), jnp.float32),
            jax.ShapeDtypeStruct((N, 16), jnp.float32),
        ),
    )(agg3, xr, wflat2, root2, bias2d)


def _fin_body(agg_ref, hr_ref, o_ref):
    v = agg_ref[0] + agg_ref[1] + hr_ref[...]
    m = jnp.max(v, axis=1, keepdims=True)
    ex = jnp.exp(v - m)
    o_ref[...] = (v - m) - jnp.log(jnp.sum(ex, axis=1, keepdims=True))


def _tc_fin(agg3, hr):
    return pl.pallas_call(
        _fin_body,
        grid=(GRID_N,),
        in_specs=[
            pl.BlockSpec((2, NB_ROWS, 16), lambda i: (0, i, 0)),
            pl.BlockSpec((NB_ROWS, 16), lambda i: (i, 0)),
        ],
        out_specs=pl.BlockSpec((NB_ROWS, 16), lambda i: (i, 0)),
        out_shape=jax.ShapeDtypeStruct((N, 16), jnp.float32),
    )(agg3, hr)


def kernel(x, edge_index, edge_type, basis1, comp1, root1, bias1,
           basis2, comp2, root2, bias2):
    src = edge_index[0]
    dst = edge_index[1]

    gidx2, cidx2 = _tc_edge(
        src.reshape(2500, 128), dst.reshape(2500, 128),
        edge_type.reshape(2500, 128))
    gidx3 = gidx2.reshape(NW, NBATCH, BT)
    cidx3 = cidx2.reshape(NW, NBATCH, BT)
    dst3 = dst.reshape(NW, NBATCH, BT)

    wflat1 = jnp.einsum("rb,bio->iro", comp1, basis1).reshape(IN_C, R * HID)
    wflat2 = jnp.einsum("rb,bio->iro", comp2, basis2).reshape(HID, R * NC)

    h1, xr1 = _tc_prep(x, wflat1, root1, bias1.reshape(1, 16))
    agg1, norm3 = _sc_msg1(gidx3, dst3, cidx3, h1.reshape(NR, 16))
    agg1 = agg1.reshape(NCORES, N_PAD, 16)

    h2, hr2 = _tc_mid(agg1, xr1, wflat2, root2, bias2.reshape(1, 16))
    agg2 = _sc_msg2(gidx3, dst3, norm3, h2.reshape(NR, 16))
    agg2 = agg2.reshape(NCORES, N_PAD, 16)

    return _tc_fin(agg2, hr2)


# speedup vs baseline: 1.3802x; 1.1080x over previous
import functools

import jax
import jax.numpy as jnp
from jax import lax
from jax.experimental import pallas as pl
from jax.experimental.pallas import tpu as pltpu
from jax.experimental.pallas import tpu_sc as plsc

N = 10000
E = 320000
IN_C = 128
HID = 16
R = 40
NC = 16
NR = N * R
NR_PAD = 409600

NCORES = 2
NSUB = 16
NW = NCORES * NSUB
EW = E // NW
BT = 80
NBATCH = EW // BT

N_PAD = 10240
ROWS_T = N_PAD // NSUB
CNT_T = NR_PAD // NSUB
CNT_CH = 3200
NB_ROWS = 2000
GRID_N = N // NB_ROWS


def _sc_mesh():
    return plsc.VectorSubcoreMesh(core_axis_name="c", subcore_axis_name="s")


_SC_PARAMS = pltpu.CompilerParams(use_tc_tiling_on_sc=False)


def _fill1d(ref, n, val):
    @pl.loop(0, n // 16)
    def _(i):
        ref[pl.ds(i * 16, 16)] = jnp.full((16,), val, ref.dtype)


def _fill2d(ref, rows, val):
    @pl.loop(0, rows)
    def _(i):
        ref[i, :] = jnp.full((16,), val, ref.dtype)


def _msg_prologue(gidx_hbm, dst_hbm, gidx_v, dst_v, tile_v, agg_sh, wid, s):
    pltpu.sync_copy(gidx_hbm.at[wid], gidx_v)
    pltpu.sync_copy(dst_hbm.at[wid], dst_v)
    _fill2d(tile_v, ROWS_T, 0.0)
    row0 = pl.multiple_of(s * ROWS_T, 8)
    pltpu.sync_copy(tile_v, agg_sh.at[pl.ds(row0, ROWS_T), :])
    return row0


def _msg_scale_scatter(rv, norm_v, j, dst_v, agg_sh):
    @pl.loop(0, BT // 16)
    def _(g):
        nv = norm_v[j, pl.ds(g * 16, 16)]
        for k in range(16):
            e = g * 16 + k
            rv[e, :] = rv[e, :] * nv[k]

    pltpu.sync_copy(rv, agg_sh.at[dst_v.at[j]], add=True)


def _msg_epilogue(agg_sh, tile_v, agg_hbm, row0, c, s):
    plsc.subcore_barrier()
    pltpu.sync_copy(agg_sh.at[pl.ds(row0, ROWS_T), :], tile_v)
    out_row = pl.multiple_of(c * N_PAD + s * ROWS_T, 8)
    pltpu.sync_copy(tile_v, agg_hbm.at[pl.ds(out_row, ROWS_T), :])


_MSG_SCRATCH1 = [
    pltpu.VMEM((NBATCH, BT), jnp.int32),
    pltpu.VMEM((NBATCH, BT), jnp.int32),
    pltpu.VMEM((NBATCH, BT), jnp.int32),
    pltpu.VMEM((NBATCH, BT), jnp.int32),
    pltpu.VMEM((NBATCH, BT), jnp.int32),
    pltpu.VMEM((NBATCH, BT), jnp.float32),
    pltpu.VMEM((BT, 16), jnp.float32),
    pltpu.VMEM((BT, 16), jnp.float32),
    pltpu.VMEM((ROWS_T, 16), jnp.float32),
    pltpu.VMEM((CNT_CH,), jnp.float32),
    pltpu.VMEM((BT,), jnp.float32),
    pltpu.VMEM_SHARED((N_PAD, 16), jnp.float32),
    pltpu.VMEM_SHARED((NR_PAD,), jnp.float32),
    pltpu.SemaphoreType.DMA,
    pltpu.SemaphoreType.DMA,
    pltpu.SemaphoreType.DMA,
    pltpu.SemaphoreType.DMA,
    pltpu.SemaphoreType.DMA,
]


@functools.partial(
    pl.kernel,
    out_type=(
        jax.ShapeDtypeStruct((NCORES * N_PAD, 16), jnp.float32),
        jax.ShapeDtypeStruct((NW, NBATCH, BT), jnp.float32),
    ),
    mesh=_sc_mesh(),
    compiler_params=_SC_PARAMS,
    scratch_types=_MSG_SCRATCH1,
)
def _sc_msg1(gidx_hbm, dst_hbm, cidx_hbm, h_hbm,
             agg_hbm, norm_out_hbm,
             gidx_v, dst_v, cidx_v, cnta_v, cntb_v, norm_v, rows0, rows1,
             tile_v, cbuf_v, ones_v, agg_sh, cnt_sh,
             gsem0, gsem1, nsem0, nsem1, csem):
    c = lax.axis_index("c")
    s = lax.axis_index("s")
    wid = s * NCORES + c

    pltpu.sync_copy(cidx_hbm.at[2 * s], cnta_v)
    pltpu.sync_copy(cidx_hbm.at[2 * s + 1], cntb_v)
    pltpu.sync_copy(cidx_hbm.at[wid], cidx_v)
    _fill1d(cbuf_v, CNT_CH, 0.0)

    @pl.loop(0, CNT_T // CNT_CH)
    def _(k):
        off = pl.multiple_of(s * CNT_T + k * CNT_CH, 8)
        pltpu.sync_copy(cbuf_v, cnt_sh.at[pl.ds(off, CNT_CH)])

    _fill1d(ones_v, BT, 1.0)
    row0 = _msg_prologue(gidx_hbm, dst_hbm, gidx_v, dst_v, tile_v, agg_sh,
                         wid, s)
    plsc.subcore_barrier()

    @pl.loop(0, NBATCH // 5)
    def _(jj):
        for u in range(5):
            pltpu.async_copy(ones_v, cnt_sh.at[cnta_v.at[jj * 5 + u]], csem,
                             add=True)
            pltpu.async_copy(ones_v, cnt_sh.at[cntb_v.at[jj * 5 + u]], csem,
                             add=True)
        for u in range(10):
            pltpu.make_async_copy(
                ones_v, cnt_sh.at[pl.ds(0, BT)], csem).wait()

    plsc.subcore_barrier()

    rows = (rows0, rows1)
    gsems = (gsem0, gsem1)
    nsems = (nsem0, nsem1)

    def issue(j, b):
        pltpu.async_copy(h_hbm.at[gidx_v.at[j]], rows[b], gsems[b])
        pltpu.async_copy(cnt_sh.at[cidx_v.at[j]], norm_v.at[j], nsems[b])

    def process(j, b):
        pltpu.make_async_copy(
            h_hbm.at[pl.ds(0, BT), :], rows[b], gsems[b]).wait()
        pltpu.make_async_copy(
            cnt_sh.at[pl.ds(0, BT)], norm_v.at[j], nsems[b]).wait()

        for g in range(BT // 16):
            cnt = norm_v[j, pl.ds(g * 16, 16)]
            norm_v[j, pl.ds(g * 16, 16)] = 1.0 / jnp.maximum(cnt, 1.0)

        _msg_scale_scatter(rows[b], norm_v, j, dst_v, agg_sh)

    issue(0, 0)

    @pl.loop(0, NBATCH - 1, step=2)
    def _(j):
        issue(j + 1, 1)
        process(j, 0)
        issue(j + 2, 0)
        process(j + 1, 1)

    process(NBATCH - 1, 0)

    pltpu.sync_copy(norm_v, norm_out_hbm.at[wid])
    _msg_epilogue(agg_sh, tile_v, agg_hbm, row0, c, s)


_MSG_SCRATCH2 = [
    pltpu.VMEM((NBATCH, BT), jnp.int32),
    pltpu.VMEM((NBATCH, BT), jnp.int32),
    pltpu.VMEM((NBATCH, BT), jnp.float32),
    pltpu.VMEM((BT, 16), jnp.float32),
    pltpu.VMEM((BT, 16), jnp.float32),
    pltpu.VMEM((ROWS_T, 16), jnp.float32),
    pltpu.VMEM_SHARED((N_PAD, 16), jnp.float32),
    pltpu.SemaphoreType.DMA,
    pltpu.SemaphoreType.DMA,
]


@functools.partial(
    pl.kernel,
    out_type=jax.ShapeDtypeStruct((NCORES * N_PAD, 16), jnp.float32),
    mesh=_sc_mesh(),
    compiler_params=_SC_PARAMS,
    scratch_types=_MSG_SCRATCH2,
)
def _sc_msg2(gidx_hbm, dst_hbm, norm_hbm, h_hbm, agg_hbm,
             gidx_v, dst_v, norm_v, rows0, rows1, tile_v, agg_sh,
             gsem0, gsem1):
    c = lax.axis_index("c")
    s = lax.axis_index("s")
    wid = s * NCORES + c

    pltpu.sync_copy(norm_hbm.at[wid], norm_v)
    row0 = _msg_prologue(gidx_hbm, dst_hbm, gidx_v, dst_v, tile_v, agg_sh,
                         wid, s)
    plsc.subcore_barrier()

    rows = (rows0, rows1)
    gsems = (gsem0, gsem1)

    def issue(j, b):
        pltpu.async_copy(h_hbm.at[gidx_v.at[j]], rows[b], gsems[b])

    def process(j, b):
        pltpu.make_async_copy(
            h_hbm.at[pl.ds(0, BT), :], rows[b], gsems[b]).wait()
        _msg_scale_scatter(rows[b], norm_v, j, dst_v, agg_sh)

    issue(0, 0)

    @pl.loop(0, NBATCH - 1, step=2)
    def _(j):
        issue(j + 1, 1)
        process(j, 0)
        issue(j + 2, 0)
        process(j + 1, 1)

    process(NBATCH - 1, 0)
    _msg_epilogue(agg_sh, tile_v, agg_hbm, row0, c, s)


def _edge_body(ei_ref, t_ref, g_ref, c_ref, d_ref):
    t = t_ref[...]
    d = ei_ref[1]
    g_ref[...] = (t // 8) * (8 * N) + ei_ref[0] * 8 + (t % 8)
    c_ref[...] = d * R + t
    d_ref[...] = d


def _tc_edge(ei3, et2):
    return pl.pallas_call(
        _edge_body,
        out_shape=(
            jax.ShapeDtypeStruct(et2.shape, jnp.int32),
            jax.ShapeDtypeStruct(et2.shape, jnp.int32),
            jax.ShapeDtypeStruct(et2.shape, jnp.int32),
        ),
    )(ei3, et2)


def _prep_body(x_ref, wf_ref, r_ref, b_ref, h_ref, xr_ref):
    xv = x_ref[...]
    wf = wf_ref[...]
    for tc in range(5):
        h_ref[tc] = jnp.dot(xv, wf[:, tc * 128:(tc + 1) * 128],
                            preferred_element_type=jnp.float32)
    xr_ref[...] = (
        jnp.dot(xv, r_ref[...], preferred_element_type=jnp.float32) + b_ref[...]
    )


def _tc_prep(x, wflat, root, bias2d):
    k = x.shape[1]
    m = wflat.shape[1]
    return pl.pallas_call(
        _prep_body,
        grid=(GRID_N,),
        in_specs=[
            pl.BlockSpec((NB_ROWS, k), lambda i: (i, 0)),
            pl.BlockSpec((k, m), lambda i: (0, 0)),
            pl.BlockSpec((k, 16), lambda i: (0, 0)),
            pl.BlockSpec((1, 16), lambda i: (0, 0)),
        ],
        out_specs=(
            pl.BlockSpec((5, NB_ROWS, 128), lambda i: (0, i, 0)),
            pl.BlockSpec((NB_ROWS, 16), lambda i: (i, 0)),
        ),
        out_shape=(
            jax.ShapeDtypeStruct((5, N, 128), jnp.float32),
            jax.ShapeDtypeStruct((N, 16), jnp.float32),
        ),
    )(x, wflat, root, bias2d)


def _mid_body(agg_ref, xr_ref, wf_ref, r_ref, b_ref, h2_ref, hr_ref):
    h = jnp.maximum(agg_ref[0] + agg_ref[1] + xr_ref[...], 0.0)
    wf = wf_ref[...]
    for tc in range(5):
        h2_ref[tc] = jnp.dot(h, wf[:, tc * 128:(tc + 1) * 128],
                             preferred_element_type=jnp.float32)
    hr_ref[...] = (
        jnp.dot(h, r_ref[...], preferred_element_type=jnp.float32) + b_ref[...]
    )


def _tc_mid(agg3, xr, wflat2, root2, bias2d):
    m = wflat2.shape[1]
    return pl.pallas_call(
        _mid_body,
        grid=(GRID_N,),
        in_specs=[
            pl.BlockSpec((2, NB_ROWS, 16), lambda i: (0, i, 0)),
            pl.BlockSpec((NB_ROWS, 16), lambda i: (i, 0)),
            pl.BlockSpec((16, m), lambda i: (0, 0)),
            pl.BlockSpec((16, 16), lambda i: (0, 0)),
            pl.BlockSpec((1, 16), lambda i: (0, 0)),
        ],
        out_specs=(
            pl.BlockSpec((5, NB_ROWS, 128), lambda i: (0, i, 0)),
            pl.BlockSpec((NB_ROWS, 16), lambda i: (i, 0)),
        ),
        out_shape=(
            jax.ShapeDtypeStruct((5, N, 128), jnp.float32),
            jax.ShapeDtypeStruct((N, 16), jnp.float32),
        ),
    )(agg3, xr, wflat2, root2, bias2d)


def _fin_body(agg_ref, hr_ref, o_ref):
    v = agg_ref[0] + agg_ref[1] + hr_ref[...]
    m = jnp.max(v, axis=1, keepdims=True)
    ex = jnp.exp(v - m)
    o_ref[...] = (v - m) - jnp.log(jnp.sum(ex, axis=1, keepdims=True))


def _tc_fin(agg3, hr):
    return pl.pallas_call(
        _fin_body,
        grid=(GRID_N,),
        in_specs=[
            pl.BlockSpec((2, NB_ROWS, 16), lambda i: (0, i, 0)),
            pl.BlockSpec((NB_ROWS, 16), lambda i: (i, 0)),
        ],
        out_specs=pl.BlockSpec((NB_ROWS, 16), lambda i: (i, 0)),
        out_shape=jax.ShapeDtypeStruct((N, 16), jnp.float32),
    )(agg3, hr)


def kernel(x, edge_index, edge_type, basis1, comp1, root1, bias1,
           basis2, comp2, root2, bias2):
    gidx2, cidx2, dst2 = _tc_edge(
        edge_index.reshape(2, 2500, 128), edge_type.reshape(2500, 128))
    gidx3 = gidx2.reshape(NW, NBATCH, BT)
    cidx3 = cidx2.reshape(NW, NBATCH, BT)
    dst3 = dst2.reshape(NW, NBATCH, BT)

    wflat1 = jnp.einsum("rb,bio->iro", comp1, basis1).reshape(IN_C, R * HID)
    wflat2 = jnp.einsum("rb,bio->iro", comp2, basis2).reshape(HID, R * NC)

    h1, xr1 = _tc_prep(x, wflat1, root1, bias1.reshape(1, 16))
    agg1, norm3 = _sc_msg1(gidx3, dst3, cidx3, h1.reshape(NR, 16))
    agg1 = agg1.reshape(NCORES, N_PAD, 16)

    h2, hr2 = _tc_mid(agg1, xr1, wflat2, root2, bias2.reshape(1, 16))
    agg2 = _sc_msg2(gidx3, dst3, norm3, h2.reshape(NR, 16))
    agg2 = agg2.reshape(NCORES, N_PAD, 16)

    return _tc_fin(agg2, hr2)


# speedup vs baseline: 1.4538x; 1.0533x over previous
import functools

import jax
import jax.numpy as jnp
from jax import lax
from jax.experimental import pallas as pl
from jax.experimental.pallas import tpu as pltpu
from jax.experimental.pallas import tpu_sc as plsc

N = 10000
E = 320000
IN_C = 128
HID = 16
R = 40
NC = 16
NR = N * R
NR_PAD = 409600

NCORES = 2
NSUB = 16
NW = NCORES * NSUB
EW = E // NW
BT = 80
NBATCH = EW // BT

N_PAD = 10240
ROWS_T = N_PAD // NSUB
CNT_T = NR_PAD // NSUB
CNT_CH = 3200
NB_ROWS = 2000
GRID_N = N // NB_ROWS


def _sc_mesh():
    return plsc.VectorSubcoreMesh(core_axis_name="c", subcore_axis_name="s")


_SC_PARAMS = pltpu.CompilerParams(use_tc_tiling_on_sc=False)


def _fill1d(ref, n, val):
    @pl.loop(0, n // 16)
    def _(i):
        ref[pl.ds(i * 16, 16)] = jnp.full((16,), val, ref.dtype)


def _fill2d(ref, rows, val):
    @pl.loop(0, rows)
    def _(i):
        ref[i, :] = jnp.full((16,), val, ref.dtype)


def _msg_prologue(gidx_hbm, dst_hbm, gidx_v, dst_v, tile_v, agg_sh, wid, s):
    pltpu.sync_copy(gidx_hbm.at[wid], gidx_v)
    pltpu.sync_copy(dst_hbm.at[wid], dst_v)
    _fill2d(tile_v, ROWS_T, 0.0)
    row0 = pl.multiple_of(s * ROWS_T, 8)
    pltpu.sync_copy(tile_v, agg_sh.at[pl.ds(row0, ROWS_T), :])
    return row0


def _msg_scale(rv, norm_v, j):
    @pl.loop(0, BT // 16)
    def _(g):
        nv = norm_v[j, pl.ds(g * 16, 16)]
        for k in range(16):
            e = g * 16 + k
            rv[e, :] = rv[e, :] * nv[k]


def _msg_epilogue(agg_sh, tile_v, agg_hbm, row0, c, s):
    plsc.subcore_barrier()
    pltpu.sync_copy(agg_sh.at[pl.ds(row0, ROWS_T), :], tile_v)
    out_row = pl.multiple_of(c * N_PAD + s * ROWS_T, 8)
    pltpu.sync_copy(tile_v, agg_hbm.at[pl.ds(out_row, ROWS_T), :])


_MSG_SCRATCH1 = [
    pltpu.VMEM((NBATCH, BT), jnp.int32),
    pltpu.VMEM((NBATCH, BT), jnp.int32),
    pltpu.VMEM((NBATCH, BT), jnp.int32),
    pltpu.VMEM((NBATCH, BT), jnp.int32),
    pltpu.VMEM((NBATCH, BT), jnp.int32),
    pltpu.VMEM((NBATCH, BT), jnp.float32),
    pltpu.VMEM((BT, 16), jnp.float32),
    pltpu.VMEM((BT, 16), jnp.float32),
    pltpu.VMEM((BT, 16), jnp.float32),
    pltpu.VMEM((BT, 16), jnp.float32),
    pltpu.VMEM((ROWS_T, 16), jnp.float32),
    pltpu.VMEM((CNT_CH,), jnp.float32),
    pltpu.VMEM((BT,), jnp.float32),
    pltpu.VMEM_SHARED((N_PAD, 16), jnp.float32),
    pltpu.VMEM_SHARED((NR_PAD,), jnp.float32),
] + [pltpu.SemaphoreType.DMA] * 13


@functools.partial(
    pl.kernel,
    out_type=(
        jax.ShapeDtypeStruct((NCORES * N_PAD, 16), jnp.float32),
        jax.ShapeDtypeStruct((NW, NBATCH, BT), jnp.float32),
    ),
    mesh=_sc_mesh(),
    compiler_params=_SC_PARAMS,
    scratch_types=_MSG_SCRATCH1,
)
def _sc_msg1(gidx_hbm, dst_hbm, cidx_hbm, h_hbm,
             agg_hbm, norm_out_hbm,
             gidx_v, dst_v, cidx_v, cnta_v, cntb_v, norm_v,
             rows0, rows1, rows2, rows3,
             tile_v, cbuf_v, ones_v, agg_sh, cnt_sh,
             gsem0, gsem1, gsem2, gsem3, nsem0, nsem1, nsem2, nsem3,
             ssem0, ssem1, ssem2, ssem3, csem):
    c = lax.axis_index("c")
    s = lax.axis_index("s")
    wid = s * NCORES + c

    pltpu.sync_copy(cidx_hbm.at[2 * s], cnta_v)
    pltpu.sync_copy(cidx_hbm.at[2 * s + 1], cntb_v)
    pltpu.sync_copy(cidx_hbm.at[wid], cidx_v)
    _fill1d(cbuf_v, CNT_CH, 0.0)

    @pl.loop(0, CNT_T // CNT_CH)
    def _(k):
        off = pl.multiple_of(s * CNT_T + k * CNT_CH, 8)
        pltpu.sync_copy(cbuf_v, cnt_sh.at[pl.ds(off, CNT_CH)])

    _fill1d(ones_v, BT, 1.0)
    row0 = _msg_prologue(gidx_hbm, dst_hbm, gidx_v, dst_v, tile_v, agg_sh,
                         wid, s)
    plsc.subcore_barrier()

    @pl.loop(0, NBATCH // 5)
    def _(jj):
        for u in range(5):
            pltpu.async_copy(ones_v, cnt_sh.at[cnta_v.at[jj * 5 + u]], csem,
                             add=True)
            pltpu.async_copy(ones_v, cnt_sh.at[cntb_v.at[jj * 5 + u]], csem,
                             add=True)
        for u in range(10):
            pltpu.make_async_copy(
                ones_v, cnt_sh.at[pl.ds(0, BT)], csem).wait()

    plsc.subcore_barrier()

    rows = (rows0, rows1, rows2, rows3)
    gsems = (gsem0, gsem1, gsem2, gsem3)
    nsems = (nsem0, nsem1, nsem2, nsem3)
    ssems = (ssem0, ssem1, ssem2, ssem3)

    def issue_g(k, b):
        pltpu.async_copy(h_hbm.at[gidx_v.at[k]], rows[b], gsems[b])
        pltpu.async_copy(cnt_sh.at[cidx_v.at[k]], norm_v.at[k], nsems[b])

    def issue_s(k, b):
        pltpu.async_copy(rows[b], agg_sh.at[dst_v.at[k]], ssems[b], add=True)

    def wait_s(b):
        pltpu.make_async_copy(
            h_hbm.at[pl.ds(0, BT), :], rows[b], ssems[b]).wait()

    def slot(k, b, pre):
        pltpu.make_async_copy(
            h_hbm.at[pl.ds(0, BT), :], rows[b], gsems[b]).wait()
        pltpu.make_async_copy(
            cnt_sh.at[pl.ds(0, BT)], norm_v.at[k], nsems[b]).wait()

        for g in range(BT // 16):
            cnt = norm_v[k, pl.ds(g * 16, 16)]
            norm_v[k, pl.ds(g * 16, 16)] = 1.0 / jnp.maximum(cnt, 1.0)

        _msg_scale(rows[b], norm_v, k)
        issue_s(k, b)
        if pre is not None:
            k2, b2, w = pre
            if w:
                wait_s(b2)
            issue_g(k2, b2)

    issue_g(0, 0)
    issue_g(1, 1)
    slot(0, 0, (2, 2, False))
    slot(1, 1, (3, 3, False))
    slot(2, 2, (4, 0, True))
    slot(3, 3, (5, 1, True))

    @pl.loop(4, NBATCH - 5, step=4)
    def _(j):
        for b in range(4):
            slot(j + b, b, (j + b + 2, (b + 2) % 4, True))

    slot(120, 0, (122, 2, True))
    slot(121, 1, (123, 3, True))
    slot(122, 2, (124, 0, True))
    slot(123, 3, None)
    slot(124, 0, None)
    wait_s(1)
    wait_s(2)
    wait_s(3)
    wait_s(0)

    pltpu.sync_copy(norm_v, norm_out_hbm.at[wid])
    _msg_epilogue(agg_sh, tile_v, agg_hbm, row0, c, s)


_MSG_SCRATCH2 = [
    pltpu.VMEM((NBATCH, BT), jnp.int32),
    pltpu.VMEM((NBATCH, BT), jnp.int32),
    pltpu.VMEM((NBATCH, BT), jnp.float32),
    pltpu.VMEM((BT, 16), jnp.float32),
    pltpu.VMEM((BT, 16), jnp.float32),
    pltpu.VMEM((BT, 16), jnp.float32),
    pltpu.VMEM((BT, 16), jnp.float32),
    pltpu.VMEM((ROWS_T, 16), jnp.float32),
    pltpu.VMEM_SHARED((N_PAD, 16), jnp.float32),
] + [pltpu.SemaphoreType.DMA] * 8


@functools.partial(
    pl.kernel,
    out_type=jax.ShapeDtypeStruct((NCORES * N_PAD, 16), jnp.float32),
    mesh=_sc_mesh(),
    compiler_params=_SC_PARAMS,
    scratch_types=_MSG_SCRATCH2,
)
def _sc_msg2(gidx_hbm, dst_hbm, norm_hbm, h_hbm, agg_hbm,
             gidx_v, dst_v, norm_v, rows0, rows1, rows2, rows3,
             tile_v, agg_sh,
             gsem0, gsem1, gsem2, gsem3, ssem0, ssem1, ssem2, ssem3):
    c = lax.axis_index("c")
    s = lax.axis_index("s")
    wid = s * NCORES + c

    pltpu.sync_copy(norm_hbm.at[wid], norm_v)
    row0 = _msg_prologue(gidx_hbm, dst_hbm, gidx_v, dst_v, tile_v, agg_sh,
                         wid, s)
    plsc.subcore_barrier()

    rows = (rows0, rows1, rows2, rows3)
    gsems = (gsem0, gsem1, gsem2, gsem3)
    ssems = (ssem0, ssem1, ssem2, ssem3)

    def issue_g(k, b):
        pltpu.async_copy(h_hbm.at[gidx_v.at[k]], rows[b], gsems[b])

    def issue_s(k, b):
        pltpu.async_copy(rows[b], agg_sh.at[dst_v.at[k]], ssems[b], add=True)

    def wait_s(b):
        pltpu.make_async_copy(
            h_hbm.at[pl.ds(0, BT), :], rows[b], ssems[b]).wait()

    def slot(k, b, pre):
        pltpu.make_async_copy(
            h_hbm.at[pl.ds(0, BT), :], rows[b], gsems[b]).wait()
        _msg_scale(rows[b], norm_v, k)
        issue_s(k, b)
        if pre is not None:
            k2, b2, w = pre
            if w:
                wait_s(b2)
            issue_g(k2, b2)

    issue_g(0, 0)
    issue_g(1, 1)
    slot(0, 0, (2, 2, False))
    slot(1, 1, (3, 3, False))
    slot(2, 2, (4, 0, True))
    slot(3, 3, (5, 1, True))

    @pl.loop(4, NBATCH - 5, step=4)
    def _(j):
        for b in range(4):
            slot(j + b, b, (j + b + 2, (b + 2) % 4, True))

    slot(120, 0, (122, 2, True))
    slot(121, 1, (123, 3, True))
    slot(122, 2, (124, 0, True))
    slot(123, 3, None)
    slot(124, 0, None)
    wait_s(1)
    wait_s(2)
    wait_s(3)
    wait_s(0)
    _msg_epilogue(agg_sh, tile_v, agg_hbm, row0, c, s)


def _edge_body(ei_ref, t_ref, g_ref, c_ref, d_ref):
    t = t_ref[...]
    d = ei_ref[1]
    g_ref[...] = (t // 8) * (8 * N) + ei_ref[0] * 8 + (t % 8)
    c_ref[...] = d * R + t
    d_ref[...] = d


def _tc_edge(ei3, et2):
    return pl.pallas_call(
        _edge_body,
        out_shape=(
            jax.ShapeDtypeStruct(et2.shape, jnp.int32),
            jax.ShapeDtypeStruct(et2.shape, jnp.int32),
            jax.ShapeDtypeStruct(et2.shape, jnp.int32),
        ),
    )(ei3, et2)


def _prep_body(x_ref, wf_ref, r_ref, b_ref, h_ref, xr_ref):
    xv = x_ref[...]
    wf = wf_ref[...]
    for tc in range(5):
        h_ref[tc] = jnp.dot(xv, wf[:, tc * 128:(tc + 1) * 128],
                            preferred_element_type=jnp.float32)
    xr_ref[...] = (
        jnp.dot(xv, r_ref[...], preferred_element_type=jnp.float32) + b_ref[...]
    )


def _tc_prep(x, wflat, root, bias2d):
    k = x.shape[1]
    m = wflat.shape[1]
    return pl.pallas_call(
        _prep_body,
        grid=(GRID_N,),
        in_specs=[
            pl.BlockSpec((NB_ROWS, k), lambda i: (i, 0)),
            pl.BlockSpec((k, m), lambda i: (0, 0)),
            pl.BlockSpec((k, 16), lambda i: (0, 0)),
            pl.BlockSpec((1, 16), lambda i: (0, 0)),
        ],
        out_specs=(
            pl.BlockSpec((5, NB_ROWS, 128), lambda i: (0, i, 0)),
            pl.BlockSpec((NB_ROWS, 16), lambda i: (i, 0)),
        ),
        out_shape=(
            jax.ShapeDtypeStruct((5, N, 128), jnp.float32),
            jax.ShapeDtypeStruct((N, 16), jnp.float32),
        ),
    )(x, wflat, root, bias2d)


def _mid_body(agg_ref, xr_ref, wf_ref, r_ref, b_ref, h2_ref, hr_ref):
    h = jnp.maximum(agg_ref[0] + agg_ref[1] + xr_ref[...], 0.0)
    wf = wf_ref[...]
    for tc in range(5):
        h2_ref[tc] = jnp.dot(h, wf[:, tc * 128:(tc + 1) * 128],
                             preferred_element_type=jnp.float32)
    hr_ref[...] = (
        jnp.dot(h, r_ref[...], preferred_element_type=jnp.float32) + b_ref[...]
    )


def _tc_mid(agg3, xr, wflat2, root2, bias2d):
    m = wflat2.shape[1]
    return pl.pallas_call(
        _mid_body,
        grid=(GRID_N,),
        in_specs=[
            pl.BlockSpec((2, NB_ROWS, 16), lambda i: (0, i, 0)),
            pl.BlockSpec((NB_ROWS, 16), lambda i: (i, 0)),
            pl.BlockSpec((16, m), lambda i: (0, 0)),
            pl.BlockSpec((16, 16), lambda i: (0, 0)),
            pl.BlockSpec((1, 16), lambda i: (0, 0)),
        ],
        out_specs=(
            pl.BlockSpec((5, NB_ROWS, 128), lambda i: (0, i, 0)),
            pl.BlockSpec((NB_ROWS, 16), lambda i: (i, 0)),
        ),
        out_shape=(
            jax.ShapeDtypeStruct((5, N, 128), jnp.float32),
            jax.ShapeDtypeStruct((N, 16), jnp.float32),
        ),
    )(agg3, xr, wflat2, root2, bias2d)


def _fin_body(agg_ref, hr_ref, o_ref):
    v = agg_ref[0] + agg_ref[1] + hr_ref[...]
    m = jnp.max(v, axis=1, keepdims=True)
    ex = jnp.exp(v - m)
    o_ref[...] = (v - m) - jnp.log(jnp.sum(ex, axis=1, keepdims=True))


def _tc_fin(agg3, hr):
    return pl.pallas_call(
        _fin_body,
        grid=(GRID_N,),
        in_specs=[
            pl.BlockSpec((2, NB_ROWS, 16), lambda i: (0, i, 0)),
            pl.BlockSpec((NB_ROWS, 16), lambda i: (i, 0)),
        ],
        out_specs=pl.BlockSpec((NB_ROWS, 16), lambda i: (i, 0)),
        out_shape=jax.ShapeDtypeStruct((N, 16), jnp.float32),
    )(agg3, hr)


def kernel(x, edge_index, edge_type, basis1, comp1, root1, bias1,
           basis2, comp2, root2, bias2):
    gidx2, cidx2, dst2 = _tc_edge(
        edge_index.reshape(2, 2500, 128), edge_type.reshape(2500, 128))
    gidx3 = gidx2.reshape(NW, NBATCH, BT)
    cidx3 = cidx2.reshape(NW, NBATCH, BT)
    dst3 = dst2.reshape(NW, NBATCH, BT)

    wflat1 = jnp.einsum("rb,bio->iro", comp1, basis1).reshape(IN_C, R * HID)
    wflat2 = jnp.einsum("rb,bio->iro", comp2, basis2).reshape(HID, R * NC)

    h1, xr1 = _tc_prep(x, wflat1, root1, bias1.reshape(1, 16))
    agg1, norm3 = _sc_msg1(gidx3, dst3, cidx3, h1.reshape(NR, 16))
    agg1 = agg1.reshape(NCORES, N_PAD, 16)

    h2, hr2 = _tc_mid(agg1, xr1, wflat2, root2, bias2.reshape(1, 16))
    agg2 = _sc_msg2(gidx3, dst3, norm3, h2.reshape(NR, 16))
    agg2 = agg2.reshape(NCORES, N_PAD, 16)

    return _tc_fin(agg2, hr2)


# speedup vs baseline: 1.8293x; 1.2583x over previous
import functools

import jax
import jax.numpy as jnp
from jax import lax
from jax.experimental import pallas as pl
from jax.experimental.pallas import tpu as pltpu
from jax.experimental.pallas import tpu_sc as plsc

N = 10000
E = 320000
IN_C = 128
HID = 16
R = 40
NC = 16
NR = N * R
NR_PAD = 409600

NCORES = 2
NSUB = 16
NW = NCORES * NSUB
EW = E // NW
BT = 80
NBATCH = EW // BT

N_PAD = 10240
ROWS_T = N_PAD // NSUB
CNT_T = NR_PAD // NSUB
CNT_CH = 3200
NB_ROWS = 2000
GRID_N = N // NB_ROWS


def _sc_mesh():
    return plsc.VectorSubcoreMesh(core_axis_name="c", subcore_axis_name="s")


_SC_PARAMS = pltpu.CompilerParams(use_tc_tiling_on_sc=False)


def _fill1d(ref, n, val):
    @pl.loop(0, n // 16)
    def _(i):
        ref[pl.ds(i * 16, 16)] = jnp.full((16,), val, ref.dtype)


def _fill2d(ref, rows, val):
    @pl.loop(0, rows)
    def _(i):
        ref[i, :] = jnp.full((16,), val, ref.dtype)


def _msg_prologue(gidx_hbm, dst_hbm, gidx_v, dst_v, tile_v, agg_sh, wid, s):
    pltpu.sync_copy(gidx_hbm.at[wid], gidx_v)
    pltpu.sync_copy(dst_hbm.at[wid], dst_v)
    _fill2d(tile_v, ROWS_T, 0.0)
    row0 = pl.multiple_of(s * ROWS_T, 8)
    pltpu.sync_copy(tile_v, agg_sh.at[pl.ds(row0, ROWS_T), :])
    return row0


def _msg_scale(rv, norm_v, j):
    @pl.loop(0, BT // 16)
    def _(g):
        nv = norm_v[j, pl.ds(g * 16, 16)]
        for k in range(16):
            e = g * 16 + k
            rv[e, :] = rv[e, :] * nv[k]


def _msg_epilogue(agg_sh, tile_v, agg_hbm, row0, c, s):
    plsc.subcore_barrier()
    pltpu.sync_copy(agg_sh.at[pl.ds(row0, ROWS_T), :], tile_v)
    out_row = pl.multiple_of(c * N_PAD + s * ROWS_T, 8)
    pltpu.sync_copy(tile_v, agg_hbm.at[pl.ds(out_row, ROWS_T), :])


_MSG_SCRATCH1 = [
    pltpu.VMEM((NBATCH, BT), jnp.int32),
    pltpu.VMEM((NBATCH, BT), jnp.int32),
    pltpu.VMEM((NBATCH, BT), jnp.int32),
    pltpu.VMEM((NBATCH, BT), jnp.int32),
    pltpu.VMEM((NBATCH, BT), jnp.int32),
    pltpu.VMEM((NBATCH, BT), jnp.float32),
] + [pltpu.VMEM((BT, 16), jnp.float32)] * 8 + [
    pltpu.VMEM((ROWS_T, 16), jnp.float32),
    pltpu.VMEM((CNT_CH,), jnp.float32),
    pltpu.VMEM((BT,), jnp.float32),
    pltpu.VMEM_SHARED((N_PAD, 16), jnp.float32),
    pltpu.VMEM_SHARED((NR_PAD,), jnp.float32),
] + [pltpu.SemaphoreType.DMA] * 25


@functools.partial(
    pl.kernel,
    out_type=(
        jax.ShapeDtypeStruct((NCORES * N_PAD, 16), jnp.float32),
        jax.ShapeDtypeStruct((NW, NBATCH, BT), jnp.float32),
    ),
    mesh=_sc_mesh(),
    compiler_params=_SC_PARAMS,
    scratch_types=_MSG_SCRATCH1,
)
def _sc_msg1(gidx_hbm, dst_hbm, cidx_hbm, h_hbm,
             agg_hbm, norm_out_hbm,
             gidx_v, dst_v, cidx_v, cnta_v, cntb_v, norm_v,
             rows0, rows1, rows2, rows3, rows4, rows5, rows6, rows7,
             tile_v, cbuf_v, ones_v, agg_sh, cnt_sh,
             gsem0, gsem1, gsem2, gsem3, gsem4, gsem5, gsem6, gsem7,
             nsem0, nsem1, nsem2, nsem3, nsem4, nsem5, nsem6, nsem7,
             ssem0, ssem1, ssem2, ssem3, ssem4, ssem5, ssem6, ssem7, csem):
    c = lax.axis_index("c")
    s = lax.axis_index("s")
    wid = s * NCORES + c

    pltpu.sync_copy(cidx_hbm.at[2 * s], cnta_v)
    pltpu.sync_copy(cidx_hbm.at[2 * s + 1], cntb_v)
    pltpu.sync_copy(cidx_hbm.at[wid], cidx_v)
    _fill1d(cbuf_v, CNT_CH, 0.0)

    @pl.loop(0, CNT_T // CNT_CH)
    def _(k):
        off = pl.multiple_of(s * CNT_T + k * CNT_CH, 8)
        pltpu.sync_copy(cbuf_v, cnt_sh.at[pl.ds(off, CNT_CH)])

    _fill1d(ones_v, BT, 1.0)
    row0 = _msg_prologue(gidx_hbm, dst_hbm, gidx_v, dst_v, tile_v, agg_sh,
                         wid, s)
    plsc.subcore_barrier()

    @pl.loop(0, NBATCH // 5)
    def _(jj):
        for u in range(5):
            pltpu.async_copy(ones_v, cnt_sh.at[cnta_v.at[jj * 5 + u]], csem,
                             add=True)
            pltpu.async_copy(ones_v, cnt_sh.at[cntb_v.at[jj * 5 + u]], csem,
                             add=True)
        for u in range(10):
            pltpu.make_async_copy(
                ones_v, cnt_sh.at[pl.ds(0, BT)], csem).wait()

    plsc.subcore_barrier()

    rows = (rows0, rows1, rows2, rows3, rows4, rows5, rows6, rows7)
    gsems = (gsem0, gsem1, gsem2, gsem3, gsem4, gsem5, gsem6, gsem7)
    nsems = (nsem0, nsem1, nsem2, nsem3, nsem4, nsem5, nsem6, nsem7)
    ssems = (ssem0, ssem1, ssem2, ssem3, ssem4, ssem5, ssem6, ssem7)

    def issue_g(k, b):
        pltpu.async_copy(h_hbm.at[gidx_v.at[k]], rows[b], gsems[b])
        pltpu.async_copy(cnt_sh.at[cidx_v.at[k]], norm_v.at[k], nsems[b])

    def issue_s(k, b):
        pltpu.async_copy(rows[b], agg_sh.at[dst_v.at[k]], ssems[b], add=True)

    def wait_s(b):
        pltpu.make_async_copy(
            h_hbm.at[pl.ds(0, BT), :], rows[b], ssems[b]).wait()

    def slot(k, b, pre):
        pltpu.make_async_copy(
            h_hbm.at[pl.ds(0, BT), :], rows[b], gsems[b]).wait()
        pltpu.make_async_copy(
            cnt_sh.at[pl.ds(0, BT)], norm_v.at[k], nsems[b]).wait()

        for g in range(BT // 16):
            cnt = norm_v[k, pl.ds(g * 16, 16)]
            norm_v[k, pl.ds(g * 16, 16)] = 1.0 / jnp.maximum(cnt, 1.0)

        _msg_scale(rows[b], norm_v, k)
        issue_s(k, b)
        if pre is not None:
            k2, b2, w = pre
            if w:
                wait_s(b2)
            issue_g(k2, b2)

    for b in range(4):
        issue_g(b, b)
    for k in range(4):
        slot(k, k, (k + 4, k + 4, False))
    for k in range(4, 8):
        slot(k, k % 8, (k + 4, (k + 4) % 8, True))

    @pl.loop(8, NBATCH - 5, step=8)
    def _(j):
        for b in range(8):
            slot(j + b, b, (j + b + 4, (b + 4) % 8, True))

    slot(120, 0, (124, 4, True))
    slot(121, 1, None)
    slot(122, 2, None)
    slot(123, 3, None)
    slot(124, 4, None)
    for b in (5, 6, 7, 0, 1, 2, 3, 4):
        wait_s(b)

    pltpu.sync_copy(norm_v, norm_out_hbm.at[wid])
    _msg_epilogue(agg_sh, tile_v, agg_hbm, row0, c, s)


_MSG_SCRATCH2 = [
    pltpu.VMEM((NBATCH, BT), jnp.int32),
    pltpu.VMEM((NBATCH, BT), jnp.int32),
    pltpu.VMEM((NBATCH, BT), jnp.float32),
] + [pltpu.VMEM((BT, 16), jnp.float32)] * 8 + [
    pltpu.VMEM((ROWS_T, 16), jnp.float32),
    pltpu.VMEM_SHARED((N_PAD, 16), jnp.float32),
] + [pltpu.SemaphoreType.DMA] * 16


@functools.partial(
    pl.kernel,
    out_type=jax.ShapeDtypeStruct((NCORES * N_PAD, 16), jnp.float32),
    mesh=_sc_mesh(),
    compiler_params=_SC_PARAMS,
    scratch_types=_MSG_SCRATCH2,
)
def _sc_msg2(gidx_hbm, dst_hbm, norm_hbm, h_hbm, agg_hbm,
             gidx_v, dst_v, norm_v,
             rows0, rows1, rows2, rows3, rows4, rows5, rows6, rows7,
             tile_v, agg_sh,
             gsem0, gsem1, gsem2, gsem3, gsem4, gsem5, gsem6, gsem7,
             ssem0, ssem1, ssem2, ssem3, ssem4, ssem5, ssem6, ssem7):
    c = lax.axis_index("c")
    s = lax.axis_index("s")
    wid = s * NCORES + c

    pltpu.sync_copy(norm_hbm.at[wid], norm_v)
    row0 = _msg_prologue(gidx_hbm, dst_hbm, gidx_v, dst_v, tile_v, agg_sh,
                         wid, s)
    plsc.subcore_barrier()

    rows = (rows0, rows1, rows2, rows3, rows4, rows5, rows6, rows7)
    gsems = (gsem0, gsem1, gsem2, gsem3, gsem4, gsem5, gsem6, gsem7)
    ssems = (ssem0, ssem1, ssem2, ssem3, ssem4, ssem5, ssem6, ssem7)

    def issue_g(k, b):
        pltpu.async_copy(h_hbm.at[gidx_v.at[k]], rows[b], gsems[b])

    def issue_s(k, b):
        pltpu.async_copy(rows[b], agg_sh.at[dst_v.at[k]], ssems[b], add=True)

    def wait_s(b):
        pltpu.make_async_copy(
            h_hbm.at[pl.ds(0, BT), :], rows[b], ssems[b]).wait()

    def slot(k, b, pre):
        pltpu.make_async_copy(
            h_hbm.at[pl.ds(0, BT), :], rows[b], gsems[b]).wait()
        _msg_scale(rows[b], norm_v, k)
        issue_s(k, b)
        if pre is not None:
            k2, b2, w = pre
            if w:
                wait_s(b2)
            issue_g(k2, b2)

    for b in range(4):
        issue_g(b, b)
    for k in range(4):
        slot(k, k, (k + 4, k + 4, False))
    for k in range(4, 8):
        slot(k, k % 8, (k + 4, (k + 4) % 8, True))

    @pl.loop(8, NBATCH - 5, step=8)
    def _(j):
        for b in range(8):
            slot(j + b, b, (j + b + 4, (b + 4) % 8, True))

    slot(120, 0, (124, 4, True))
    slot(121, 1, None)
    slot(122, 2, None)
    slot(123, 3, None)
    slot(124, 4, None)
    for b in (5, 6, 7, 0, 1, 2, 3, 4):
        wait_s(b)
    _msg_epilogue(agg_sh, tile_v, agg_hbm, row0, c, s)


def _edge_body(ei_ref, t_ref, g_ref, c_ref, d_ref):
    t = t_ref[...]
    d = ei_ref[1]
    g_ref[...] = (t // 8) * (8 * N) + ei_ref[0] * 8 + (t % 8)
    c_ref[...] = d * R + t
    d_ref[...] = d


def _tc_edge(ei3, et2):
    return pl.pallas_call(
        _edge_body,
        out_shape=(
            jax.ShapeDtypeStruct(et2.shape, jnp.int32),
            jax.ShapeDtypeStruct(et2.shape, jnp.int32),
            jax.ShapeDtypeStruct(et2.shape, jnp.int32),
        ),
    )(ei3, et2)


def _prep_body(x_ref, wf_ref, r_ref, b_ref, h_ref, xr_ref):
    xv = x_ref[...]
    wf = wf_ref[...]
    for tc in range(5):
        h_ref[tc] = jnp.dot(xv, wf[:, tc * 128:(tc + 1) * 128],
                            preferred_element_type=jnp.float32)
    xr_ref[...] = (
        jnp.dot(xv, r_ref[...], preferred_element_type=jnp.float32) + b_ref[...]
    )


def _tc_prep(x, wflat, root, bias2d):
    k = x.shape[1]
    m = wflat.shape[1]
    return pl.pallas_call(
        _prep_body,
        grid=(GRID_N,),
        in_specs=[
            pl.BlockSpec((NB_ROWS, k), lambda i: (i, 0)),
            pl.BlockSpec((k, m), lambda i: (0, 0)),
            pl.BlockSpec((k, 16), lambda i: (0, 0)),
            pl.BlockSpec((1, 16), lambda i: (0, 0)),
        ],
        out_specs=(
            pl.BlockSpec((5, NB_ROWS, 128), lambda i: (0, i, 0)),
            pl.BlockSpec((NB_ROWS, 16), lambda i: (i, 0)),
        ),
        out_shape=(
            jax.ShapeDtypeStruct((5, N, 128), jnp.float32),
            jax.ShapeDtypeStruct((N, 16), jnp.float32),
        ),
    )(x, wflat, root, bias2d)


def _mid_body(agg_ref, xr_ref, wf_ref, r_ref, b_ref, h2_ref, hr_ref):
    h = jnp.maximum(agg_ref[0] + agg_ref[1] + xr_ref[...], 0.0)
    wf = wf_ref[...]
    for tc in range(5):
        h2_ref[tc] = jnp.dot(h, wf[:, tc * 128:(tc + 1) * 128],
                             preferred_element_type=jnp.float32)
    hr_ref[...] = (
        jnp.dot(h, r_ref[...], preferred_element_type=jnp.float32) + b_ref[...]
    )


def _tc_mid(agg3, xr, wflat2, root2, bias2d):
    m = wflat2.shape[1]
    return pl.pallas_call(
        _mid_body,
        grid=(GRID_N,),
        in_specs=[
            pl.BlockSpec((2, NB_ROWS, 16), lambda i: (0, i, 0)),
            pl.BlockSpec((NB_ROWS, 16), lambda i: (i, 0)),
            pl.BlockSpec((16, m), lambda i: (0, 0)),
            pl.BlockSpec((16, 16), lambda i: (0, 0)),
            pl.BlockSpec((1, 16), lambda i: (0, 0)),
        ],
        out_specs=(
            pl.BlockSpec((5, NB_ROWS, 128), lambda i: (0, i, 0)),
            pl.BlockSpec((NB_ROWS, 16), lambda i: (i, 0)),
        ),
        out_shape=(
            jax.ShapeDtypeStruct((5, N, 128), jnp.float32),
            jax.ShapeDtypeStruct((N, 16), jnp.float32),
        ),
    )(agg3, xr, wflat2, root2, bias2d)


def _fin_body(agg_ref, hr_ref, o_ref):
    v = agg_ref[0] + agg_ref[1] + hr_ref[...]
    m = jnp.max(v, axis=1, keepdims=True)
    ex = jnp.exp(v - m)
    o_ref[...] = (v - m) - jnp.log(jnp.sum(ex, axis=1, keepdims=True))


def _tc_fin(agg3, hr):
    return pl.pallas_call(
        _fin_body,
        grid=(GRID_N,),
        in_specs=[
            pl.BlockSpec((2, NB_ROWS, 16), lambda i: (0, i, 0)),
            pl.BlockSpec((NB_ROWS, 16), lambda i: (i, 0)),
        ],
        out_specs=pl.BlockSpec((NB_ROWS, 16), lambda i: (i, 0)),
        out_shape=jax.ShapeDtypeStruct((N, 16), jnp.float32),
    )(agg3, hr)


def kernel(x, edge_index, edge_type, basis1, comp1, root1, bias1,
           basis2, comp2, root2, bias2):
    gidx2, cidx2, dst2 = _tc_edge(
        edge_index.reshape(2, 2500, 128), edge_type.reshape(2500, 128))
    gidx3 = gidx2.reshape(NW, NBATCH, BT)
    cidx3 = cidx2.reshape(NW, NBATCH, BT)
    dst3 = dst2.reshape(NW, NBATCH, BT)

    wflat1 = jnp.einsum("rb,bio->iro", comp1, basis1).reshape(IN_C, R * HID)
    wflat2 = jnp.einsum("rb,bio->iro", comp2, basis2).reshape(HID, R * NC)

    h1, xr1 = _tc_prep(x, wflat1, root1, bias1.reshape(1, 16))
    agg1, norm3 = _sc_msg1(gidx3, dst3, cidx3, h1.reshape(NR, 16))
    agg1 = agg1.reshape(NCORES, N_PAD, 16)

    h2, hr2 = _tc_mid(agg1, xr1, wflat2, root2, bias2.reshape(1, 16))
    agg2 = _sc_msg2(gidx3, dst3, norm3, h2.reshape(NR, 16))
    agg2 = agg2.reshape(NCORES, N_PAD, 16)

    return _tc_fin(agg2, hr2)


# speedup vs baseline: 1.8975x; 1.0372x over previous
import functools

import jax
import jax.numpy as jnp
from jax import lax
from jax.experimental import pallas as pl
from jax.experimental.pallas import tpu as pltpu
from jax.experimental.pallas import tpu_sc as plsc

N = 10000
E = 320000
IN_C = 128
HID = 16
R = 40
NC = 16
NR = N * R
NR_PAD = 409600

NCORES = 2
NSUB = 16
NW = NCORES * NSUB
EW = E // NW
BT = 80
NBATCH = EW // BT

N_PAD = 10240
ROWS_T = N_PAD // NSUB
CNT_T = NR_PAD // NSUB
CNT_CH = 3200
NB_ROWS = 2000
GRID_N = N // NB_ROWS


def _sc_mesh():
    return plsc.VectorSubcoreMesh(core_axis_name="c", subcore_axis_name="s")


_SC_PARAMS = pltpu.CompilerParams(use_tc_tiling_on_sc=False)


def _fill1d(ref, n, val):
    @pl.loop(0, n // 16)
    def _(i):
        ref[pl.ds(i * 16, 16)] = jnp.full((16,), val, ref.dtype)


def _fill2d(ref, rows, val):
    @pl.loop(0, rows)
    def _(i):
        ref[i, :] = jnp.full((16,), val, ref.dtype)


def _msg_prologue(gidx_hbm, dst_hbm, gidx_v, dst_v, tile_v, agg_sh, wid, s):
    pltpu.sync_copy(gidx_hbm.at[wid], gidx_v)
    pltpu.sync_copy(dst_hbm.at[wid], dst_v)
    _fill2d(tile_v, ROWS_T, 0.0)
    row0 = pl.multiple_of(s * ROWS_T, 8)
    pltpu.sync_copy(tile_v, agg_sh.at[pl.ds(row0, ROWS_T), :])
    return row0


def _msg_scale(rv, norm_v, j):
    @pl.loop(0, BT // 16)
    def _(g):
        nv = norm_v[j, pl.ds(g * 16, 16)]
        for k in range(16):
            e = g * 16 + k
            rv[e, :] = rv[e, :] * nv[k]


def _msg_epilogue(agg_sh, tile_v, agg_hbm, row0, c, s):
    plsc.subcore_barrier()
    pltpu.sync_copy(agg_sh.at[pl.ds(row0, ROWS_T), :], tile_v)
    out_row = pl.multiple_of(c * N_PAD + s * ROWS_T, 8)
    pltpu.sync_copy(tile_v, agg_hbm.at[pl.ds(out_row, ROWS_T), :])


_MSG_SCRATCH1 = [
    pltpu.VMEM((NBATCH, BT), jnp.int32),
    pltpu.VMEM((NBATCH, BT), jnp.int32),
    pltpu.VMEM((NBATCH, BT), jnp.int32),
    pltpu.VMEM((NBATCH, BT), jnp.int32),
    pltpu.VMEM((NBATCH, BT), jnp.int32),
    pltpu.VMEM((NBATCH, BT), jnp.float32),
] + [pltpu.VMEM((BT, 16), jnp.float32)] * 8 + [
    pltpu.VMEM((ROWS_T, 16), jnp.float32),
    pltpu.VMEM((CNT_CH,), jnp.float32),
    pltpu.VMEM((BT,), jnp.float32),
    pltpu.VMEM_SHARED((N_PAD, 16), jnp.float32),
    pltpu.VMEM_SHARED((NR_PAD,), jnp.float32),
] + [pltpu.SemaphoreType.DMA] * 25


@functools.partial(
    pl.kernel,
    out_type=(
        jax.ShapeDtypeStruct((NCORES * N_PAD, 16), jnp.float32),
        jax.ShapeDtypeStruct((NW, NBATCH, BT), jnp.float32),
    ),
    mesh=_sc_mesh(),
    compiler_params=_SC_PARAMS,
    scratch_types=_MSG_SCRATCH1,
)
def _sc_msg1(gidx_hbm, dst_hbm, cidx_hbm, h_hbm,
             agg_hbm, norm_out_hbm,
             gidx_v, dst_v, cidx_v, cnta_v, cntb_v, norm_v,
             rows0, rows1, rows2, rows3, rows4, rows5, rows6, rows7,
             tile_v, cbuf_v, ones_v, agg_sh, cnt_sh,
             gsem0, gsem1, gsem2, gsem3, gsem4, gsem5, gsem6, gsem7,
             nsem0, nsem1, nsem2, nsem3, nsem4, nsem5, nsem6, nsem7,
             ssem0, ssem1, ssem2, ssem3, ssem4, ssem5, ssem6, ssem7, csem):
    c = lax.axis_index("c")
    s = lax.axis_index("s")
    wid = s * NCORES + c

    pltpu.sync_copy(cidx_hbm.at[2 * s], cnta_v)
    pltpu.sync_copy(cidx_hbm.at[2 * s + 1], cntb_v)
    pltpu.sync_copy(cidx_hbm.at[wid], cidx_v)
    _fill1d(cbuf_v, CNT_CH, 0.0)

    @pl.loop(0, CNT_T // CNT_CH)
    def _(k):
        off = pl.multiple_of(s * CNT_T + k * CNT_CH, 8)
        pltpu.sync_copy(cbuf_v, cnt_sh.at[pl.ds(off, CNT_CH)])

    _fill1d(ones_v, BT, 1.0)
    row0 = _msg_prologue(gidx_hbm, dst_hbm, gidx_v, dst_v, tile_v, agg_sh,
                         wid, s)
    plsc.subcore_barrier()

    for u in range(5):
        pltpu.async_copy(ones_v, cnt_sh.at[cnta_v.at[u]], csem, add=True)
        pltpu.async_copy(ones_v, cnt_sh.at[cntb_v.at[u]], csem, add=True)

    @pl.loop(1, NBATCH // 5)
    def _(jj):
        for u in range(5):
            pltpu.async_copy(ones_v, cnt_sh.at[cnta_v.at[jj * 5 + u]], csem,
                             add=True)
            pltpu.async_copy(ones_v, cnt_sh.at[cntb_v.at[jj * 5 + u]], csem,
                             add=True)
        for u in range(10):
            pltpu.make_async_copy(
                ones_v, cnt_sh.at[pl.ds(0, BT)], csem).wait()

    for u in range(10):
        pltpu.make_async_copy(ones_v, cnt_sh.at[pl.ds(0, BT)], csem).wait()

    plsc.subcore_barrier()

    rows = (rows0, rows1, rows2, rows3, rows4, rows5, rows6, rows7)
    gsems = (gsem0, gsem1, gsem2, gsem3, gsem4, gsem5, gsem6, gsem7)
    nsems = (nsem0, nsem1, nsem2, nsem3, nsem4, nsem5, nsem6, nsem7)
    ssems = (ssem0, ssem1, ssem2, ssem3, ssem4, ssem5, ssem6, ssem7)

    def issue_g(k, b):
        pltpu.async_copy(h_hbm.at[gidx_v.at[k]], rows[b], gsems[b])
        pltpu.async_copy(cnt_sh.at[cidx_v.at[k]], norm_v.at[k], nsems[b])

    def issue_s(k, b):
        pltpu.async_copy(rows[b], agg_sh.at[dst_v.at[k]], ssems[b], add=True)

    def wait_s(b):
        pltpu.make_async_copy(
            h_hbm.at[pl.ds(0, BT), :], rows[b], ssems[b]).wait()

    def slot(k, b, pre):
        pltpu.make_async_copy(
            h_hbm.at[pl.ds(0, BT), :], rows[b], gsems[b]).wait()
        pltpu.make_async_copy(
            cnt_sh.at[pl.ds(0, BT)], norm_v.at[k], nsems[b]).wait()

        for g in range(BT // 16):
            cnt = norm_v[k, pl.ds(g * 16, 16)]
            norm_v[k, pl.ds(g * 16, 16)] = 1.0 / jnp.maximum(cnt, 1.0)

        _msg_scale(rows[b], norm_v, k)
        issue_s(k, b)
        if pre is not None:
            k2, b2, w = pre
            if w:
                wait_s(b2)
            issue_g(k2, b2)

    for b in range(4):
        issue_g(b, b)
    for k in range(4):
        slot(k, k, (k + 4, k + 4, False))
    for k in range(4, 8):
        slot(k, k % 8, (k + 4, (k + 4) % 8, True))

    @pl.loop(8, NBATCH - 5, step=8)
    def _(j):
        for b in range(8):
            slot(j + b, b, (j + b + 4, (b + 4) % 8, True))

    slot(120, 0, (124, 4, True))
    slot(121, 1, None)
    slot(122, 2, None)
    slot(123, 3, None)
    slot(124, 4, None)
    for b in (5, 6, 7, 0, 1, 2, 3, 4):
        wait_s(b)

    pltpu.sync_copy(norm_v, norm_out_hbm.at[wid])
    _msg_epilogue(agg_sh, tile_v, agg_hbm, row0, c, s)


_MSG_SCRATCH2 = [
    pltpu.VMEM((NBATCH, BT), jnp.int32),
    pltpu.VMEM((NBATCH, BT), jnp.int32),
    pltpu.VMEM((NBATCH, BT), jnp.float32),
] + [pltpu.VMEM((BT, 16), jnp.float32)] * 8 + [
    pltpu.VMEM((ROWS_T, 16), jnp.float32),
    pltpu.VMEM_SHARED((N_PAD, 16), jnp.float32),
] + [pltpu.SemaphoreType.DMA] * 16


@functools.partial(
    pl.kernel,
    out_type=jax.ShapeDtypeStruct((NCORES * N_PAD, 16), jnp.float32),
    mesh=_sc_mesh(),
    compiler_params=_SC_PARAMS,
    scratch_types=_MSG_SCRATCH2,
)
def _sc_msg2(gidx_hbm, dst_hbm, norm_hbm, h_hbm, agg_hbm,
             gidx_v, dst_v, norm_v,
             rows0, rows1, rows2, rows3, rows4, rows5, rows6, rows7,
             tile_v, agg_sh,
             gsem0, gsem1, gsem2, gsem3, gsem4, gsem5, gsem6, gsem7,
             ssem0, ssem1, ssem2, ssem3, ssem4, ssem5, ssem6, ssem7):
    c = lax.axis_index("c")
    s = lax.axis_index("s")
    wid = s * NCORES + c

    pltpu.sync_copy(norm_hbm.at[wid], norm_v)
    row0 = _msg_prologue(gidx_hbm, dst_hbm, gidx_v, dst_v, tile_v, agg_sh,
                         wid, s)
    plsc.subcore_barrier()

    rows = (rows0, rows1, rows2, rows3, rows4, rows5, rows6, rows7)
    gsems = (gsem0, gsem1, gsem2, gsem3, gsem4, gsem5, gsem6, gsem7)
    ssems = (ssem0, ssem1, ssem2, ssem3, ssem4, ssem5, ssem6, ssem7)

    def issue_g(k, b):
        pltpu.async_copy(h_hbm.at[gidx_v.at[k]], rows[b], gsems[b])

    def issue_s(k, b):
        pltpu.async_copy(rows[b], agg_sh.at[dst_v.at[k]], ssems[b], add=True)

    def wait_s(b):
        pltpu.make_async_copy(
            h_hbm.at[pl.ds(0, BT), :], rows[b], ssems[b]).wait()

    def slot(k, b, pre):
        pltpu.make_async_copy(
            h_hbm.at[pl.ds(0, BT), :], rows[b], gsems[b]).wait()
        _msg_scale(rows[b], norm_v, k)
        issue_s(k, b)
        if pre is not None:
            k2, b2, w = pre
            if w:
                wait_s(b2)
            issue_g(k2, b2)

    for b in range(4):
        issue_g(b, b)
    for k in range(4):
        slot(k, k, (k + 4, k + 4, False))
    for k in range(4, 8):
        slot(k, k % 8, (k + 4, (k + 4) % 8, True))

    @pl.loop(8, NBATCH - 5, step=8)
    def _(j):
        for b in range(8):
            slot(j + b, b, (j + b + 4, (b + 4) % 8, True))

    slot(120, 0, (124, 4, True))
    slot(121, 1, None)
    slot(122, 2, None)
    slot(123, 3, None)
    slot(124, 4, None)
    for b in (5, 6, 7, 0, 1, 2, 3, 4):
        wait_s(b)
    _msg_epilogue(agg_sh, tile_v, agg_hbm, row0, c, s)


def _edge_body(ei_ref, t_ref, g_ref, c_ref, d_ref):
    t = t_ref[...]
    d = ei_ref[1]
    g_ref[...] = (t // 8) * (8 * N) + ei_ref[0] * 8 + (t % 8)
    c_ref[...] = d * R + t
    d_ref[...] = d


def _tc_edge(ei, et):
    return pl.pallas_call(
        _edge_body,
        out_shape=(
            jax.ShapeDtypeStruct((E,), jnp.int32),
            jax.ShapeDtypeStruct((E,), jnp.int32),
            jax.ShapeDtypeStruct((E,), jnp.int32),
        ),
    )(ei, et)


def _prep_body(x_ref, wf_ref, r_ref, b_ref, h_ref, xr_ref):
    xv = x_ref[...]
    wf = wf_ref[...]
    for tc in range(5):
        h_ref[tc] = jnp.dot(xv, wf[:, tc * 128:(tc + 1) * 128],
                            preferred_element_type=jnp.float32)
    xr_ref[...] = (
        jnp.dot(xv, r_ref[...], preferred_element_type=jnp.float32) + b_ref[...]
    )


def _tc_prep(x, wflat, root, bias2d):
    k = x.shape[1]
    m = wflat.shape[1]
    return pl.pallas_call(
        _prep_body,
        grid=(GRID_N,),
        in_specs=[
            pl.BlockSpec((NB_ROWS, k), lambda i: (i, 0)),
            pl.BlockSpec((k, m), lambda i: (0, 0)),
            pl.BlockSpec((k, 16), lambda i: (0, 0)),
            pl.BlockSpec((1, 16), lambda i: (0, 0)),
        ],
        out_specs=(
            pl.BlockSpec((5, NB_ROWS, 128), lambda i: (0, i, 0)),
            pl.BlockSpec((NB_ROWS, 16), lambda i: (i, 0)),
        ),
        out_shape=(
            jax.ShapeDtypeStruct((5, N, 128), jnp.float32),
            jax.ShapeDtypeStruct((N, 16), jnp.float32),
        ),
    )(x, wflat, root, bias2d)


def _mid_body(agg_ref, xr_ref, wf_ref, r_ref, b_ref, h2_ref, hr_ref):
    h = jnp.maximum(agg_ref[0] + agg_ref[1] + xr_ref[...], 0.0)
    wf = wf_ref[...]
    for tc in range(5):
        h2_ref[tc] = jnp.dot(h, wf[:, tc * 128:(tc + 1) * 128],
                             preferred_element_type=jnp.float32)
    hr_ref[...] = (
        jnp.dot(h, r_ref[...], preferred_element_type=jnp.float32) + b_ref[...]
    )


def _tc_mid(agg3, xr, wflat2, root2, bias2d):
    m = wflat2.shape[1]
    return pl.pallas_call(
        _mid_body,
        grid=(GRID_N,),
        in_specs=[
            pl.BlockSpec((2, NB_ROWS, 16), lambda i: (0, i, 0)),
            pl.BlockSpec((NB_ROWS, 16), lambda i: (i, 0)),
            pl.BlockSpec((16, m), lambda i: (0, 0)),
            pl.BlockSpec((16, 16), lambda i: (0, 0)),
            pl.BlockSpec((1, 16), lambda i: (0, 0)),
        ],
        out_specs=(
            pl.BlockSpec((5, NB_ROWS, 128), lambda i: (0, i, 0)),
            pl.BlockSpec((NB_ROWS, 16), lambda i: (i, 0)),
        ),
        out_shape=(
            jax.ShapeDtypeStruct((5, N, 128), jnp.float32),
            jax.ShapeDtypeStruct((N, 16), jnp.float32),
        ),
    )(agg3, xr, wflat2, root2, bias2d)


def _fin_body(agg_ref, hr_ref, o_ref):
    v = agg_ref[0] + agg_ref[1] + hr_ref[...]
    m = jnp.max(v, axis=1, keepdims=True)
    ex = jnp.exp(v - m)
    o_ref[...] = (v - m) - jnp.log(jnp.sum(ex, axis=1, keepdims=True))


def _tc_fin(agg3, hr):
    return pl.pallas_call(
        _fin_body,
        grid=(GRID_N,),
        in_specs=[
            pl.BlockSpec((2, NB_ROWS, 16), lambda i: (0, i, 0)),
            pl.BlockSpec((NB_ROWS, 16), lambda i: (i, 0)),
        ],
        out_specs=pl.BlockSpec((NB_ROWS, 16), lambda i: (i, 0)),
        out_shape=jax.ShapeDtypeStruct((N, 16), jnp.float32),
    )(agg3, hr)


def kernel(x, edge_index, edge_type, basis1, comp1, root1, bias1,
           basis2, comp2, root2, bias2):
    gidx1, cidx1, dst1 = _tc_edge(edge_index, edge_type)
    gidx3 = gidx1.reshape(NW, NBATCH, BT)
    cidx3 = cidx1.reshape(NW, NBATCH, BT)
    dst3 = dst1.reshape(NW, NBATCH, BT)

    wflat1 = jnp.einsum("rb,bio->iro", comp1, basis1).reshape(IN_C, R * HID)
    wflat2 = jnp.einsum("rb,bio->iro", comp2, basis2).reshape(HID, R * NC)

    h1, xr1 = _tc_prep(x, wflat1, root1, bias1.reshape(1, 16))
    agg1, norm3 = _sc_msg1(gidx3, dst3, cidx3, h1.reshape(NR, 16))
    agg1 = agg1.reshape(NCORES, N_PAD, 16)

    h2, hr2 = _tc_mid(agg1, xr1, wflat2, root2, bias2.reshape(1, 16))
    agg2 = _sc_msg2(gidx3, dst3, norm3, h2.reshape(NR, 16))
    agg2 = agg2.reshape(NCORES, N_PAD, 16)

    return _tc_fin(agg2, hr2)
